# baseline (speedup 1.0000x reference)
import functools

import jax
import jax.numpy as jnp
from jax import lax
from jax.experimental import pallas as pl
from jax.experimental.pallas import tpu as pltpu

_B = 4
_N = 2048
_C = 80
_MAXP = 300
_KPAD = 304
_CONF_TH = 0.25
_IOU_TH = 0.1
_SIGMA = 2.0
_EPS = 1e-7
_TR = 256
_NB = _N // _TR
_CW = 2 * _TR
_NPAD = _N + _TR
_NEG = -1e30
_BIG = 1e30


def _row_params(bTc):
    xr = bTc[0:1, :]
    yr = bTc[1:2, :]
    wr = bTc[2:3, :]
    hr = bTc[3:4, :]
    rr = bTc[4:5, :]
    ar = wr * wr / 12.0
    br = hr * hr / 12.0
    cosr = jnp.cos(rr)
    sinr = jnp.sin(rr)
    Ar = ar * cosr * cosr + br * sinr * sinr
    Br = ar * sinr * sinr + br * cosr * cosr
    Cr = (ar - br) * cosr * sinr
    det = jnp.clip(Ar * Br - Cr * Cr, _EPS, None)
    rsd = 1.0 / jnp.sqrt(det)
    return xr, yr, Ar, Br, Cr, rsd


def _nms_body(bxT_ref, scT_ref, nk_ref, bb_ref, ss_ref, cc_ref,
              comp_ref, strict_ref):
    k = pl.program_id(1)

    @pl.when(k == 0)
    def _init():
        comp_ref[...] = jnp.zeros((1, _NPAD), jnp.float32)
        strict_ref[...] = jnp.full((1, _NPAD), _NEG, jnp.float32)

    bTk = bxT_ref[0, :, pl.ds(k * _TR, _TR)]
    xk, yk, Ak, Bk, Ck, rsdk = _row_params(bTk)
    zpad = jnp.zeros((2, _TR), jnp.float32)
    rowstack = jnp.concatenate([xk, yk, Ak, Bk, Ck, rsdk, zpad], axis=0)
    colstack = lax.transpose(rowstack, (1, 0))
    xc = colstack[:, 0:1]
    yc = colstack[:, 1:2]
    Ac = colstack[:, 2:3]
    Bc = colstack[:, 3:4]
    Cc = colstack[:, 4:5]
    rsdc = colstack[:, 5:6]

    def iou_tile(start, w):
        bTc = bxT_ref[0, :, pl.ds(start, w)]
        xr, yr, Ar, Br, Cr, rsdr = _row_params(bTc)
        rsdrq = 0.25 * rsdr
        As = Ac + Ar
        Bs = Bc + Br
        Cs = Cc + Cr
        denom = As * Bs - Cs * Cs + _EPS
        rden = 1.0 / denom
        dy = yc - yr
        dx = xc - xr
        t12 = (0.25 * (As * dy * dy + Bs * dx * dx)
               - 0.5 * Cs * dx * dy) * rden
        t3 = 0.5 * jnp.log(denom * (rsdc * rsdrq) + _EPS)
        bd = jnp.clip(t12 + t3, _EPS, 100.0)
        hd = jnp.sqrt(1.0 - jnp.exp(-bd) + _EPS)
        return 1.0 - hd

    li = lax.broadcasted_iota(jnp.int32, (_TR, _TR), 0)
    lj = lax.broadcasted_iota(jnp.int32, (_TR, _TR), 1)
    dmask = li < lj
    iou_d = iou_tile(k * _TR, _TR)
    iou_dm = jnp.where(dmask, iou_d, 0.0)
    dsl = pl.ds(k * _TR, _TR)
    comp_k = jnp.maximum(comp_ref[0:1, dsl],
                         jnp.max(iou_dm, axis=0, keepdims=True))
    comp_ref[0:1, dsl] = comp_k

    compcol = jnp.max(jnp.where(li == lj, comp_k, _NEG), axis=1,
                      keepdims=True)
    csq = compcol * compcol

    term_d = jnp.where(dmask, iou_d * iou_d - csq, _NEG)
    strict_ref[0:1, dsl] = jnp.maximum(
        strict_ref[0:1, dsl], jnp.max(term_d, axis=0, keepdims=True))

    def chunk(t, _):
        start = (k + 1) * _TR + t * _CW
        sl = pl.ds(start, _CW)
        iou = iou_tile(start, _CW)
        comp_ref[0:1, sl] = jnp.maximum(
            comp_ref[0:1, sl], jnp.max(iou, axis=0, keepdims=True))
        strict_ref[0:1, sl] = jnp.maximum(
            strict_ref[0:1, sl],
            jnp.max(iou * iou - csq, axis=0, keepdims=True))
        return 0

    lax.fori_loop(0, (_NB - k) // 2, chunk, 0)

    @pl.when(k == _NB - 1)
    def _epilogue():
        comp = comp_ref[0:1, :_N]
        x = comp
        sh = 1
        while sh < _N:
            shifted = jnp.concatenate(
                [x[:, sh:], jnp.full((1, sh), _BIG, jnp.float32)], axis=1)
            x = jnp.minimum(x, shifted)
            sh *= 2
        suffmin = x
        decayarg = jnp.maximum(strict_ref[0:1, :_N], -(suffmin * suffmin))
        decay = jnp.exp(-_SIGMA * decayarg)

        sc = scT_ref[0]
        confr = jnp.max(sc, axis=0, keepdims=True)
        idxc = lax.broadcasted_iota(jnp.int32, (_C, _N), 0)
        lab = jnp.min(jnp.where(sc == confr, idxc, 2 ** 30), axis=0,
                      keepdims=True)
        confr = jnp.where(confr < _CONF_TH, 0.0, confr)
        keep = (confr * decay) > _IOU_TH
        kf = keep.astype(jnp.float32)

        y = kf
        sh = 1
        while sh < _N:
            shifted = jnp.concatenate(
                [jnp.zeros((1, sh), jnp.float32), y[:, :_N - sh]], axis=1)
            y = y + shifted
            sh *= 2
        ck = y - kf
        nk = jnp.sum(kf)
        jr = lax.broadcasted_iota(jnp.int32, (1, _N), 1).astype(jnp.float32)
        pos = jnp.where(keep, ck, nk + (jr - ck))

        rowid = lax.broadcasted_iota(jnp.int32, (_KPAD, _N),
                                     0).astype(jnp.float32)
        oh = (rowid == pos).astype(jnp.float32)
        nt = (((1,), (1,)), ((), ()))
        hi = lax.Precision.HIGHEST
        bT = bxT_ref[0, :, : _N]
        bout = lax.dot_general(oh, bT, nt, precision=hi,
                               preferred_element_type=jnp.float32)
        sout = lax.dot_general(oh, confr, nt, precision=hi,
                               preferred_element_type=jnp.float32)
        cout = lax.dot_general(oh, lab.astype(jnp.float32), nt, precision=hi,
                               preferred_element_type=jnp.float32)

        kidx = lax.broadcasted_iota(jnp.int32, (_KPAD, 1),
                                    0).astype(jnp.float32)
        valid = kidx < jnp.minimum(nk, float(_MAXP))
        bb_ref[0] = jnp.where(valid, bout, -1.0)
        ss_ref[0] = jnp.where(valid, sout, -1.0)
        cc_ref[0] = jnp.where(valid, cout.astype(jnp.int32), -1)
        nk_ref[0] = nk.astype(jnp.int32).reshape(1, 1)


@jax.jit
def kernel(pred_boxes, pred_scores):
    boxesT = pred_boxes.transpose(0, 2, 1)
    boxesT = jnp.pad(boxesT, ((0, 0), (0, 0), (0, _NPAD - _N)))
    scoresT = pred_scores.transpose(0, 2, 1)

    grid = (_B, _NB)
    out = pl.pallas_call(
        _nms_body,
        grid=grid,
        in_specs=[
            pl.BlockSpec((1, 5, _NPAD), lambda b, k: (b, 0, 0)),
            pl.BlockSpec((1, _C, _N), lambda b, k: (b, 0, 0)),
        ],
        out_specs=[
            pl.BlockSpec((1, 1, 1), lambda b, k: (b, 0, 0)),
            pl.BlockSpec((1, _KPAD, 5), lambda b, k: (b, 0, 0)),
            pl.BlockSpec((1, _KPAD, 1), lambda b, k: (b, 0, 0)),
            pl.BlockSpec((1, _KPAD, 1), lambda b, k: (b, 0, 0)),
        ],
        out_shape=[
            jax.ShapeDtypeStruct((_B, 1, 1), jnp.int32),
            jax.ShapeDtypeStruct((_B, _KPAD, 5), jnp.float32),
            jax.ShapeDtypeStruct((_B, _KPAD, 1), jnp.float32),
            jax.ShapeDtypeStruct((_B, _KPAD, 1), jnp.int32),
        ],
        scratch_shapes=[
            pltpu.VMEM((1, _NPAD), jnp.float32),
            pltpu.VMEM((1, _NPAD), jnp.float32),
        ],
        compiler_params=pltpu.CompilerParams(
            dimension_semantics=("parallel", "arbitrary")),
    )(boxesT, scoresT)
    nk3, b3, s3, c3 = out
    return (nk3.reshape(_B, 1), b3[:, :_MAXP, :],
            s3[:, :_MAXP, 0], c3[:, :_MAXP, 0])


# speedup vs baseline: 1.1495x; 1.1495x over previous
import functools

import jax
import jax.numpy as jnp
from jax import lax
from jax.experimental import pallas as pl
from jax.experimental.pallas import tpu as pltpu

_B = 4
_N = 2048
_C = 80
_MAXP = 300
_KPAD = 304
_CONF_TH = 0.25
_IOU_TH = 0.1
_SIGMA = 2.0
_EPS = 1e-7
_TR = 256
_NB = _N // _TR
_CW = 2 * _TR
_NPAD = _N + _TR
_NEG = -1e30
_BIG = 1e30


def _row_params(bTc):
    xr = bTc[0:1, :]
    yr = bTc[1:2, :]
    wr = bTc[2:3, :]
    hr = bTc[3:4, :]
    rr = bTc[4:5, :]
    ar = wr * wr / 12.0
    br = hr * hr / 12.0
    cosr = jnp.cos(rr)
    sinr = jnp.sin(rr)
    Ar = ar * cosr * cosr + br * sinr * sinr
    Br = ar * sinr * sinr + br * cosr * cosr
    Cr = (ar - br) * cosr * sinr
    det = jnp.clip(Ar * Br - Cr * Cr, _EPS, None)
    rsd = 1.0 / jnp.sqrt(det)
    return xr, yr, Ar, Br, Cr, rsd


def _nms_body(bxT_ref, scT_ref, nk_ref, bb_ref, ss_ref, cc_ref,
              comp_ref, strict_ref):
    k = pl.program_id(1)

    @pl.when(k == 0)
    def _init():
        comp_ref[...] = jnp.zeros((1, _NPAD), jnp.float32)
        strict_ref[...] = jnp.full((1, _NPAD), _NEG, jnp.float32)

    bTk = bxT_ref[0, :, pl.ds(k * _TR, _TR)]
    xk, yk, Ak, Bk, Ck, rsdk = _row_params(bTk)
    zpad = jnp.zeros((2, _TR), jnp.float32)
    rowstack = jnp.concatenate([xk, yk, Ak, Bk, Ck, rsdk, zpad], axis=0)
    colstack = lax.transpose(rowstack, (1, 0))
    xc = colstack[:, 0:1]
    yc = colstack[:, 1:2]
    Ac = colstack[:, 2:3]
    Bc = colstack[:, 3:4]
    Cc = colstack[:, 4:5]
    rsdc = colstack[:, 5:6]

    def iou_tile(start, w):
        bTc = bxT_ref[0, :, pl.ds(start, w)]
        xr, yr, Ar, Br, Cr, rsdr = _row_params(bTc)
        rsdrq = 0.25 * rsdr
        As = Ac + Ar
        Bs = Bc + Br
        Cs = Cc + Cr
        denom = As * Bs - Cs * Cs + _EPS
        rden = 1.0 / denom
        dy = yc - yr
        dx = xc - xr
        t12 = (0.25 * (As * dy * dy + Bs * dx * dx)
               - 0.5 * Cs * dx * dy) * rden
        t3 = 0.5 * jnp.log(denom * (rsdc * rsdrq) + _EPS)
        bd = jnp.clip(t12 + t3, _EPS, 100.0)
        hd = jnp.sqrt(1.0 - jnp.exp(-bd) + _EPS)
        return 1.0 - hd

    li = lax.broadcasted_iota(jnp.int32, (_TR, _TR), 0)
    lj = lax.broadcasted_iota(jnp.int32, (_TR, _TR), 1)
    dmask = li < lj
    iou_d = iou_tile(k * _TR, _TR)
    iou_dm = jnp.where(dmask, iou_d, 0.0)
    dsl = pl.ds(k * _TR, _TR)
    comp_k = jnp.maximum(comp_ref[0:1, dsl],
                         jnp.max(iou_dm, axis=0, keepdims=True))
    comp_ref[0:1, dsl] = comp_k

    compcol = jnp.max(jnp.where(li == lj, comp_k, _NEG), axis=1,
                      keepdims=True)
    csq = compcol * compcol

    term_d = jnp.where(dmask, iou_d * iou_d - csq, _NEG)
    strict_ref[0:1, dsl] = jnp.maximum(
        strict_ref[0:1, dsl], jnp.max(term_d, axis=0, keepdims=True))

    def chunk(c, _):
        start = c * _TR
        sl = pl.ds(start, _TR)
        iou = iou_tile(start, _TR)
        comp_ref[0:1, sl] = jnp.maximum(
            comp_ref[0:1, sl], jnp.max(iou, axis=0, keepdims=True))
        strict_ref[0:1, sl] = jnp.maximum(
            strict_ref[0:1, sl],
            jnp.max(iou * iou - csq, axis=0, keepdims=True))
        return 0

    lax.fori_loop(k + 1, _NB, chunk, 0)

    @pl.when(k == _NB - 1)
    def _epilogue():
        comp = comp_ref[0:1, :_N]
        x = comp
        sh = 1
        while sh < _N:
            shifted = jnp.concatenate(
                [x[:, sh:], jnp.full((1, sh), _BIG, jnp.float32)], axis=1)
            x = jnp.minimum(x, shifted)
            sh *= 2
        suffmin = x
        decayarg = jnp.maximum(strict_ref[0:1, :_N], -(suffmin * suffmin))
        decay = jnp.exp(-_SIGMA * decayarg)

        sc = scT_ref[0]
        confr = jnp.max(sc, axis=0, keepdims=True)
        idxc = lax.broadcasted_iota(jnp.int32, (_C, _N), 0)
        lab = jnp.min(jnp.where(sc == confr, idxc, 2 ** 30), axis=0,
                      keepdims=True)
        confr = jnp.where(confr < _CONF_TH, 0.0, confr)
        keep = (confr * decay) > _IOU_TH
        kf = keep.astype(jnp.float32)

        y = kf
        sh = 1
        while sh < _N:
            shifted = jnp.concatenate(
                [jnp.zeros((1, sh), jnp.float32), y[:, :_N - sh]], axis=1)
            y = y + shifted
            sh *= 2
        ck = y - kf
        nk = jnp.sum(kf)
        jr = lax.broadcasted_iota(jnp.int32, (1, _N), 1).astype(jnp.float32)
        pos = jnp.where(keep, ck, nk + (jr - ck))

        rowid = lax.broadcasted_iota(jnp.int32, (_KPAD, _N),
                                     0).astype(jnp.float32)
        oh = (rowid == pos).astype(jnp.float32)
        nt = (((1,), (1,)), ((), ()))
        hi = lax.Precision.HIGHEST
        bT = bxT_ref[0, :, : _N]
        bout = lax.dot_general(oh, bT, nt, precision=hi,
                               preferred_element_type=jnp.float32)
        sout = lax.dot_general(oh, confr, nt, precision=hi,
                               preferred_element_type=jnp.float32)
        cout = lax.dot_general(oh, lab.astype(jnp.float32), nt, precision=hi,
                               preferred_element_type=jnp.float32)

        kidx = lax.broadcasted_iota(jnp.int32, (_KPAD, 1),
                                    0).astype(jnp.float32)
        valid = kidx < jnp.minimum(nk, float(_MAXP))
        bb_ref[0] = jnp.where(valid, bout, -1.0)
        ss_ref[0] = jnp.where(valid, sout, -1.0)
        cc_ref[0] = jnp.where(valid, cout.astype(jnp.int32), -1)
        nk_ref[0] = nk.astype(jnp.int32).reshape(1, 1)


@jax.jit
def kernel(pred_boxes, pred_scores):
    boxesT = pred_boxes.transpose(0, 2, 1)
    boxesT = jnp.pad(boxesT, ((0, 0), (0, 0), (0, _NPAD - _N)))
    scoresT = pred_scores.transpose(0, 2, 1)

    grid = (_B, _NB)
    out = pl.pallas_call(
        _nms_body,
        grid=grid,
        in_specs=[
            pl.BlockSpec((1, 5, _NPAD), lambda b, k: (b, 0, 0)),
            pl.BlockSpec((1, _C, _N), lambda b, k: (b, 0, 0)),
        ],
        out_specs=[
            pl.BlockSpec((1, 1, 1), lambda b, k: (b, 0, 0)),
            pl.BlockSpec((1, _KPAD, 5), lambda b, k: (b, 0, 0)),
            pl.BlockSpec((1, _KPAD, 1), lambda b, k: (b, 0, 0)),
            pl.BlockSpec((1, _KPAD, 1), lambda b, k: (b, 0, 0)),
        ],
        out_shape=[
            jax.ShapeDtypeStruct((_B, 1, 1), jnp.int32),
            jax.ShapeDtypeStruct((_B, _KPAD, 5), jnp.float32),
            jax.ShapeDtypeStruct((_B, _KPAD, 1), jnp.float32),
            jax.ShapeDtypeStruct((_B, _KPAD, 1), jnp.int32),
        ],
        scratch_shapes=[
            pltpu.VMEM((1, _NPAD), jnp.float32),
            pltpu.VMEM((1, _NPAD), jnp.float32),
        ],
        compiler_params=pltpu.CompilerParams(
            dimension_semantics=("parallel", "arbitrary")),
    )(boxesT, scoresT)
    nk3, b3, s3, c3 = out
    return (nk3.reshape(_B, 1), b3[:, :_MAXP, :],
            s3[:, :_MAXP, 0], c3[:, :_MAXP, 0])


# speedup vs baseline: 1.2799x; 1.1134x over previous
import functools

import jax
import jax.numpy as jnp
from jax import lax
from jax.experimental import pallas as pl
from jax.experimental.pallas import tpu as pltpu

_B = 4
_N = 2048
_C = 80
_MAXP = 300
_KPAD = 304
_CONF_TH = 0.25
_IOU_TH = 0.1
_SIGMA = 2.0
_EPS = 1e-7
_TR = 256
_NB = _N // _TR
_CW = 128
_NPAD = _N
_NEG = -1e30
_BIG = 1e30


def _row_params(bTc):
    xr = bTc[0:1, :]
    yr = bTc[1:2, :]
    wr = bTc[2:3, :]
    hr = bTc[3:4, :]
    rr = bTc[4:5, :]
    ar = wr * wr / 12.0
    br = hr * hr / 12.0
    cosr = jnp.cos(rr)
    sinr = jnp.sin(rr)
    Ar = ar * cosr * cosr + br * sinr * sinr
    Br = ar * sinr * sinr + br * cosr * cosr
    Cr = (ar - br) * cosr * sinr
    det = jnp.clip(Ar * Br - Cr * Cr, _EPS, None)
    rsd = 1.0 / jnp.sqrt(det)
    return xr, yr, Ar, Br, Cr, rsd


def _nms_body(bxT_ref, scT_ref, nk_ref, bb_ref, ss_ref, cc_ref,
              comp_ref, strict_ref):
    k = pl.program_id(1)

    @pl.when(k == 0)
    def _init():
        comp_ref[...] = jnp.zeros((1, _NPAD), jnp.float32)
        strict_ref[...] = jnp.full((1, _NPAD), _NEG, jnp.float32)

    bTk = bxT_ref[0, :, pl.ds(k * _TR, _TR)]
    xk, yk, Ak, Bk, Ck, rsdk = _row_params(bTk)
    zpad = jnp.zeros((2, _TR), jnp.float32)
    rowstack = jnp.concatenate([xk, yk, Ak, Bk, Ck, rsdk, zpad], axis=0)
    colstack = lax.transpose(rowstack, (1, 0))
    xc = colstack[:, 0:1]
    yc = colstack[:, 1:2]
    Ac = colstack[:, 2:3]
    Bc = colstack[:, 3:4]
    Cc = colstack[:, 4:5]
    rsdc = colstack[:, 5:6]

    def iou_tile(start, w):
        bTc = bxT_ref[0, :, pl.ds(start, w)]
        xr, yr, Ar, Br, Cr, rsdr = _row_params(bTc)
        rsdrq = 0.25 * rsdr
        As = Ac + Ar
        Bs = Bc + Br
        Cs = Cc + Cr
        denom = As * Bs - Cs * Cs + _EPS
        rden = 1.0 / denom
        dy = yc - yr
        dx = xc - xr
        t12 = (0.25 * (As * dy * dy + Bs * dx * dx)
               - 0.5 * Cs * dx * dy) * rden
        t3 = 0.5 * jnp.log(denom * (rsdc * rsdrq) + _EPS)
        bd = jnp.maximum(t12 + t3, _EPS)
        hd = jnp.sqrt((1.0 + _EPS) - jnp.exp(-bd))
        return 1.0 - hd

    li = lax.broadcasted_iota(jnp.int32, (_TR, _TR), 0)
    lj = lax.broadcasted_iota(jnp.int32, (_TR, _TR), 1)
    dmask = li < lj
    iou_d = iou_tile(k * _TR, _TR)
    iou_dm = jnp.where(dmask, iou_d, 0.0)
    dsl = pl.ds(k * _TR, _TR)
    comp_k = jnp.maximum(comp_ref[0:1, dsl],
                         jnp.max(iou_dm, axis=0, keepdims=True))
    comp_ref[0:1, dsl] = comp_k

    compcol = jnp.max(jnp.where(li == lj, comp_k, _NEG), axis=1,
                      keepdims=True)
    csq = compcol * compcol

    term_d = jnp.where(dmask, iou_d * iou_d - csq, _NEG)
    strict_ref[0:1, dsl] = jnp.maximum(
        strict_ref[0:1, dsl], jnp.max(term_d, axis=0, keepdims=True))

    def chunk(c, _):
        start = c * _CW
        sl = pl.ds(start, _CW)
        iou = iou_tile(start, _CW)
        comp_ref[0:1, sl] = jnp.maximum(
            comp_ref[0:1, sl], jnp.max(iou, axis=0, keepdims=True))
        strict_ref[0:1, sl] = jnp.maximum(
            strict_ref[0:1, sl],
            jnp.max(iou * iou - csq, axis=0, keepdims=True))
        return 0

    lax.fori_loop((k + 1) * (_TR // _CW), _N // _CW, chunk, 0)

    @pl.when(k == _NB - 1)
    def _epilogue():
        comp = comp_ref[0:1, :_N]
        x = comp
        sh = 1
        while sh < _N:
            shifted = jnp.concatenate(
                [x[:, sh:], jnp.full((1, sh), _BIG, jnp.float32)], axis=1)
            x = jnp.minimum(x, shifted)
            sh *= 2
        suffmin = x
        decayarg = jnp.maximum(strict_ref[0:1, :_N], -(suffmin * suffmin))
        decay = jnp.exp(-_SIGMA * decayarg)

        sc = scT_ref[0]
        confr = jnp.max(sc, axis=0, keepdims=True)
        idxc = lax.broadcasted_iota(jnp.int32, (_C, _N), 0)
        lab = jnp.min(jnp.where(sc == confr, idxc, 2 ** 30), axis=0,
                      keepdims=True)
        confr = jnp.where(confr < _CONF_TH, 0.0, confr)
        keep = (confr * decay) > _IOU_TH
        kf = keep.astype(jnp.float32)

        y = kf
        sh = 1
        while sh < _N:
            shifted = jnp.concatenate(
                [jnp.zeros((1, sh), jnp.float32), y[:, :_N - sh]], axis=1)
            y = y + shifted
            sh *= 2
        ck = y - kf
        nk = jnp.sum(kf)
        jr = lax.broadcasted_iota(jnp.int32, (1, _N), 1).astype(jnp.float32)
        pos = jnp.where(keep, ck, nk + (jr - ck))

        rowid = lax.broadcasted_iota(jnp.int32, (_KPAD, _N),
                                     0).astype(jnp.float32)
        oh = (rowid == pos).astype(jnp.float32)
        nt = (((1,), (1,)), ((), ()))

        def oh_gather(vals):
            vhi = vals.astype(jnp.bfloat16).astype(jnp.float32)
            vlo = vals - vhi
            return (lax.dot_general(oh, vhi, nt,
                                    preferred_element_type=jnp.float32)
                    + lax.dot_general(oh, vlo, nt,
                                      preferred_element_type=jnp.float32))

        bT = bxT_ref[0, :, : _N]
        bout = oh_gather(bT)
        sout = oh_gather(confr)
        cout = lax.dot_general(oh, lab.astype(jnp.float32), nt,
                               preferred_element_type=jnp.float32)

        kidx = lax.broadcasted_iota(jnp.int32, (_KPAD, 1),
                                    0).astype(jnp.float32)
        valid = kidx < jnp.minimum(nk, float(_MAXP))
        bb_ref[0] = jnp.where(valid, bout, -1.0)
        ss_ref[0] = jnp.where(valid, sout, -1.0)
        cc_ref[0] = jnp.where(valid, cout.astype(jnp.int32), -1)
        nk_ref[0] = nk.astype(jnp.int32).reshape(1, 1)


@jax.jit
def kernel(pred_boxes, pred_scores):
    boxesT = pred_boxes.transpose(0, 2, 1)
    scoresT = pred_scores.transpose(0, 2, 1)

    grid = (_B, _NB)
    out = pl.pallas_call(
        _nms_body,
        grid=grid,
        in_specs=[
            pl.BlockSpec((1, 5, _NPAD), lambda b, k: (b, 0, 0)),
            pl.BlockSpec((1, _C, _N), lambda b, k: (b, 0, 0)),
        ],
        out_specs=[
            pl.BlockSpec((1, 1, 1), lambda b, k: (b, 0, 0)),
            pl.BlockSpec((1, _KPAD, 5), lambda b, k: (b, 0, 0)),
            pl.BlockSpec((1, _KPAD, 1), lambda b, k: (b, 0, 0)),
            pl.BlockSpec((1, _KPAD, 1), lambda b, k: (b, 0, 0)),
        ],
        out_shape=[
            jax.ShapeDtypeStruct((_B, 1, 1), jnp.int32),
            jax.ShapeDtypeStruct((_B, _KPAD, 5), jnp.float32),
            jax.ShapeDtypeStruct((_B, _KPAD, 1), jnp.float32),
            jax.ShapeDtypeStruct((_B, _KPAD, 1), jnp.int32),
        ],
        scratch_shapes=[
            pltpu.VMEM((1, _NPAD), jnp.float32),
            pltpu.VMEM((1, _NPAD), jnp.float32),
        ],
        compiler_params=pltpu.CompilerParams(
            dimension_semantics=("parallel", "arbitrary")),
    )(boxesT, scoresT)
    nk3, b3, s3, c3 = out
    return (nk3.reshape(_B, 1), b3[:, :_MAXP, :],
            s3[:, :_MAXP, 0], c3[:, :_MAXP, 0])


# speedup vs baseline: 1.3108x; 1.0241x over previous
import functools

import jax
import jax.numpy as jnp
from jax import lax
from jax.experimental import pallas as pl
from jax.experimental.pallas import tpu as pltpu

_B = 4
_N = 2048
_C = 80
_MAXP = 300
_KPAD = 304
_CONF_TH = 0.25
_IOU_TH = 0.1
_SIGMA = 2.0
_EPS = 1e-7
_TR = 256
_NB = _N // _TR
_CW = 128
_NPAD = _N
_NEG = -1e30
_BIG = 1e30


def _row_params(bTc):
    xr = bTc[0:1, :]
    yr = bTc[1:2, :]
    wr = bTc[2:3, :]
    hr = bTc[3:4, :]
    rr = bTc[4:5, :]
    ar = wr * wr / 12.0
    br = hr * hr / 12.0
    cosr = jnp.cos(rr)
    sinr = jnp.sin(rr)
    Ar = ar * cosr * cosr + br * sinr * sinr
    Br = ar * sinr * sinr + br * cosr * cosr
    Cr = (ar - br) * cosr * sinr
    det = jnp.clip(Ar * Br - Cr * Cr, _EPS, None)
    rsd = 1.0 / jnp.sqrt(det)
    return xr, yr, Ar, Br, Cr, rsd


def _nms_body(bxT_ref, scT_ref, nk_ref, bb_ref, ss_ref, cc_ref,
              comp_ref, strict_ref):
    k = pl.program_id(1)

    @pl.when(k == 0)
    def _init():
        comp_ref[...] = jnp.zeros((1, _NPAD), jnp.float32)
        strict_ref[...] = jnp.full((1, _NPAD), _NEG, jnp.float32)

    bTk = bxT_ref[0, :, pl.ds(k * _TR, _TR)]
    xk, yk, Ak, Bk, Ck, rsdk = _row_params(bTk)
    zpad = jnp.zeros((2, _TR), jnp.float32)
    rowstack = jnp.concatenate([xk, yk, Ak, Bk, Ck, rsdk, zpad], axis=0)
    colstack = lax.transpose(rowstack, (1, 0))
    xc = colstack[:, 0:1]
    yc = colstack[:, 1:2]
    Ac = colstack[:, 2:3]
    Bc = colstack[:, 3:4]
    Cc = colstack[:, 4:5]
    rsdc = colstack[:, 5:6]

    def iou_tile(start, w):
        bTc = bxT_ref[0, :, pl.ds(start, w)]
        xr, yr, Ar, Br, Cr, rsdr = _row_params(bTc)
        rsdrq = 0.25 * rsdr
        As = Ac + Ar
        Bs = Bc + Br
        Cs = Cc + Cr
        denom = As * Bs - Cs * Cs + _EPS
        rden = 1.0 / denom
        dy = yc - yr
        dx = xc - xr
        t12 = (0.25 * (As * dy * dy + Bs * dx * dx)
               - 0.5 * Cs * dx * dy) * rden
        t3 = 0.5 * jnp.log(denom * (rsdc * rsdrq) + _EPS)
        bd = jnp.maximum(t12 + t3, _EPS)
        hd = jnp.sqrt((1.0 + _EPS) - jnp.exp(-bd))
        return 1.0 - hd

    li = lax.broadcasted_iota(jnp.int32, (_TR, _TR), 0)
    lj = lax.broadcasted_iota(jnp.int32, (_TR, _TR), 1)
    dmask = li < lj
    iou_d = iou_tile(k * _TR, _TR)
    iou_dm = jnp.where(dmask, iou_d, 0.0)
    dsl = pl.ds(k * _TR, _TR)
    comp_k = jnp.maximum(comp_ref[0:1, dsl],
                         jnp.max(iou_dm, axis=0, keepdims=True))
    comp_ref[0:1, dsl] = comp_k

    compcol = jnp.max(jnp.where(li == lj, comp_k, _NEG), axis=1,
                      keepdims=True)
    csq = compcol * compcol

    term_d = jnp.where(dmask, iou_d * iou_d - csq, _NEG)
    strict_ref[0:1, dsl] = jnp.maximum(
        strict_ref[0:1, dsl], jnp.max(term_d, axis=0, keepdims=True))

    def chunk(c, _):
        start = c * _CW
        sl = pl.ds(start, _CW)
        iou = iou_tile(start, _CW)
        comp_ref[0:1, sl] = jnp.maximum(
            comp_ref[0:1, sl], jnp.max(iou, axis=0, keepdims=True))
        strict_ref[0:1, sl] = jnp.maximum(
            strict_ref[0:1, sl],
            jnp.max(iou * iou - csq, axis=0, keepdims=True))
        return 0

    lax.fori_loop((k + 1) * (_TR // _CW), _N // _CW, chunk, 0)

    @pl.when(k == _NB - 1)
    def _epilogue():
        comp = comp_ref[0:1, :_N]
        x = comp
        sh = 1
        while sh < _N:
            shifted = jnp.concatenate(
                [x[:, sh:], jnp.full((1, sh), _BIG, jnp.float32)], axis=1)
            x = jnp.minimum(x, shifted)
            sh *= 2
        suffmin = x
        decayarg = jnp.maximum(strict_ref[0:1, :_N], -(suffmin * suffmin))
        decay = jnp.exp(-_SIGMA * decayarg)

        sc = scT_ref[0]
        confr = jnp.max(sc, axis=0, keepdims=True)
        idxc = lax.broadcasted_iota(jnp.int32, (_C, _N), 0)
        lab = jnp.min(jnp.where(sc == confr, idxc, 2 ** 30), axis=0,
                      keepdims=True)
        confr = jnp.where(confr < _CONF_TH, 0.0, confr)
        keep = (confr * decay) > _IOU_TH
        kf = keep.astype(jnp.float32)

        y = kf
        sh = 1
        while sh < _N:
            shifted = jnp.concatenate(
                [jnp.zeros((1, sh), jnp.float32), y[:, :_N - sh]], axis=1)
            y = y + shifted
            sh *= 2
        ck = y - kf
        nk = jnp.sum(kf)
        jr = lax.broadcasted_iota(jnp.int32, (1, _N), 1).astype(jnp.float32)
        pos = jnp.where(keep, ck, nk + (jr - ck))

        rowid = lax.broadcasted_iota(jnp.int32, (_KPAD, _N),
                                     0).astype(jnp.float32)
        oh = (rowid == pos).astype(jnp.float32)
        nt = (((1,), (1,)), ((), ()))
        bT = bxT_ref[0, :, : _N]
        vals = jnp.concatenate([bT, confr], axis=0)
        vhi = vals.astype(jnp.bfloat16).astype(jnp.float32)
        vlo = vals - vhi
        rhs = jnp.concatenate([vhi, vlo, lab.astype(jnp.float32)], axis=0)
        gat = lax.dot_general(oh, rhs, nt,
                              preferred_element_type=jnp.float32)
        bout = gat[:, 0:5] + gat[:, 6:11]
        sout = gat[:, 5:6] + gat[:, 11:12]
        cout = gat[:, 12:13]

        kidx = lax.broadcasted_iota(jnp.int32, (_KPAD, 1),
                                    0).astype(jnp.float32)
        valid = kidx < jnp.minimum(nk, float(_MAXP))
        bb_ref[0] = jnp.where(valid, bout, -1.0)
        ss_ref[0] = jnp.where(valid, sout, -1.0)
        cc_ref[0] = jnp.where(valid, cout.astype(jnp.int32), -1)
        nk_ref[0] = nk.astype(jnp.int32).reshape(1, 1)


@jax.jit
def kernel(pred_boxes, pred_scores):
    boxesT = pred_boxes.transpose(0, 2, 1)
    scoresT = pred_scores.transpose(0, 2, 1)

    grid = (_B, _NB)
    out = pl.pallas_call(
        _nms_body,
        grid=grid,
        in_specs=[
            pl.BlockSpec((1, 5, _NPAD), lambda b, k: (b, 0, 0)),
            pl.BlockSpec((1, _C, _N), lambda b, k: (b, 0, 0)),
        ],
        out_specs=[
            pl.BlockSpec((1, 1, 1), lambda b, k: (b, 0, 0)),
            pl.BlockSpec((1, _KPAD, 5), lambda b, k: (b, 0, 0)),
            pl.BlockSpec((1, _KPAD, 1), lambda b, k: (b, 0, 0)),
            pl.BlockSpec((1, _KPAD, 1), lambda b, k: (b, 0, 0)),
        ],
        out_shape=[
            jax.ShapeDtypeStruct((_B, 1, 1), jnp.int32),
            jax.ShapeDtypeStruct((_B, _KPAD, 5), jnp.float32),
            jax.ShapeDtypeStruct((_B, _KPAD, 1), jnp.float32),
            jax.ShapeDtypeStruct((_B, _KPAD, 1), jnp.int32),
        ],
        scratch_shapes=[
            pltpu.VMEM((1, _NPAD), jnp.float32),
            pltpu.VMEM((1, _NPAD), jnp.float32),
        ],
        compiler_params=pltpu.CompilerParams(
            dimension_semantics=("parallel", "arbitrary")),
    )(boxesT, scoresT)
    nk3, b3, s3, c3 = out
    return (nk3.reshape(_B, 1), b3[:, :_MAXP, :],
            s3[:, :_MAXP, 0], c3[:, :_MAXP, 0])


# speedup vs baseline: 1.4331x; 1.0933x over previous
import functools

import jax
import jax.numpy as jnp
from jax import lax
from jax.experimental import pallas as pl
from jax.experimental.pallas import tpu as pltpu

_B = 4
_N = 2048
_C = 80
_MAXP = 300
_KPAD = 304
_CONF_TH = 0.25
_IOU_TH = 0.1
_SIGMA = 2.0
_EPS = 1e-7
_TR = 256
_NB = _N // _TR
_CW = 128
_NPAD = _N
_NEG = -1e30
_BIG = 1e30


def _row_params(bTc):
    xr = bTc[0:1, :]
    yr = bTc[1:2, :]
    wr = bTc[2:3, :]
    hr = bTc[3:4, :]
    rr = bTc[4:5, :]
    ar = wr * wr / 12.0
    br = hr * hr / 12.0
    cosr = jnp.cos(rr)
    sinr = jnp.sin(rr)
    Ar = ar * cosr * cosr + br * sinr * sinr
    Br = ar * sinr * sinr + br * cosr * cosr
    Cr = (ar - br) * cosr * sinr
    det = jnp.clip(Ar * Br - Cr * Cr, _EPS, None)
    rsdh = 0.5 / jnp.sqrt(det)
    return xr, yr, Ar, Br, Cr, rsdh


def _nms_body(bxT_ref, scT_ref, nk_ref, bb_ref, ss_ref, cc_ref,
              comp_ref, strict_ref, parm_ref):
    k = pl.program_id(1)

    @pl.when(k == 0)
    def _init():
        comp_ref[...] = jnp.zeros((1, _NPAD), jnp.float32)
        strict_ref[...] = jnp.full((1, _NPAD), _NEG, jnp.float32)
        xr, yr, Ar, Br, Cr, rsdh = _row_params(bxT_ref[0])
        parm_ref[...] = jnp.concatenate(
            [xr, yr, Ar, Br, Cr, rsdh, jnp.zeros((2, _N), jnp.float32)],
            axis=0)

    colstack = lax.transpose(parm_ref[:, pl.ds(k * _TR, _TR)], (1, 0))
    xc = colstack[:, 0:1]
    yc = colstack[:, 1:2]
    Ac = colstack[:, 2:3]
    Bc = colstack[:, 3:4]
    Cc = colstack[:, 4:5]
    rsdc = colstack[:, 5:6]

    def iou_tile(start, w):
        sl = pl.ds(start, w)
        xr = parm_ref[0:1, sl]
        yr = parm_ref[1:2, sl]
        Ar = parm_ref[2:3, sl]
        Br = parm_ref[3:4, sl]
        Cr = parm_ref[4:5, sl]
        rsdr = parm_ref[5:6, sl]
        As = Ac + Ar
        Bs = Bc + Br
        Cs = Cc + Cr
        denom = As * Bs - Cs * Cs + _EPS
        rden = 1.0 / denom
        dy = yc - yr
        dx = xc - xr
        t12 = (0.25 * (As * dy * dy + Bs * dx * dx)
               - 0.5 * Cs * dx * dy) * rden
        t3 = 0.5 * jnp.log(denom * (rsdc * rsdr) + _EPS)
        bd = jnp.maximum(t12 + t3, _EPS)
        hd = jnp.sqrt((1.0 + _EPS) - jnp.exp(-bd))
        return 1.0 - hd

    li = lax.broadcasted_iota(jnp.int32, (_TR, _TR), 0)
    lj = lax.broadcasted_iota(jnp.int32, (_TR, _TR), 1)
    dmask = li < lj
    iou_d = iou_tile(k * _TR, _TR)
    iou_dm = jnp.where(dmask, iou_d, 0.0)
    dsl = pl.ds(k * _TR, _TR)
    comp_k = jnp.maximum(comp_ref[0:1, dsl],
                         jnp.max(iou_dm, axis=0, keepdims=True))
    comp_ref[0:1, dsl] = comp_k

    compcol = jnp.max(jnp.where(li == lj, comp_k, _NEG), axis=1,
                      keepdims=True)
    csq = compcol * compcol

    term_d = jnp.where(dmask, iou_d * iou_d - csq, _NEG)
    strict_ref[0:1, dsl] = jnp.maximum(
        strict_ref[0:1, dsl], jnp.max(term_d, axis=0, keepdims=True))

    def half(start):
        sl = pl.ds(start, _CW)
        iou = iou_tile(start, _CW)
        comp_ref[0:1, sl] = jnp.maximum(
            comp_ref[0:1, sl], jnp.max(iou, axis=0, keepdims=True))
        strict_ref[0:1, sl] = jnp.maximum(
            strict_ref[0:1, sl],
            jnp.max(iou * iou - csq, axis=0, keepdims=True))

    def chunk(c, _):
        half(c * _TR)
        half(c * _TR + _CW)
        return 0

    lax.fori_loop(k + 1, _NB, chunk, 0)

    @pl.when(k == _NB - 1)
    def _epilogue():
        comp = comp_ref[0:1, :_N]
        x = comp
        sh = 1
        while sh < _N:
            shifted = jnp.concatenate(
                [x[:, sh:], jnp.full((1, sh), _BIG, jnp.float32)], axis=1)
            x = jnp.minimum(x, shifted)
            sh *= 2
        suffmin = x
        decayarg = jnp.maximum(strict_ref[0:1, :_N], -(suffmin * suffmin))
        decay = jnp.exp(-_SIGMA * decayarg)

        sc = scT_ref[0]
        confr = jnp.max(sc, axis=0, keepdims=True)
        idxc = lax.broadcasted_iota(jnp.int32, (_C, _N), 0)
        lab = jnp.min(jnp.where(sc == confr, idxc, 2 ** 30), axis=0,
                      keepdims=True)
        confr = jnp.where(confr < _CONF_TH, 0.0, confr)
        keep = (confr * decay) > _IOU_TH
        kf = keep.astype(jnp.float32)

        y = kf
        sh = 1
        while sh < _N:
            shifted = jnp.concatenate(
                [jnp.zeros((1, sh), jnp.float32), y[:, :_N - sh]], axis=1)
            y = y + shifted
            sh *= 2
        ck = y - kf
        nk = jnp.sum(kf)
        jr = lax.broadcasted_iota(jnp.int32, (1, _N), 1).astype(jnp.float32)
        pos = jnp.where(keep, ck, nk + (jr - ck))

        rowid = lax.broadcasted_iota(jnp.int32, (_KPAD, _N),
                                     0).astype(jnp.float32)
        oh = (rowid == pos).astype(jnp.float32)
        nt = (((1,), (1,)), ((), ()))
        bT = bxT_ref[0, :, : _N]
        vals = jnp.concatenate([bT, confr], axis=0)
        vhi = vals.astype(jnp.bfloat16).astype(jnp.float32)
        vlo = vals - vhi
        rhs = jnp.concatenate([vhi, vlo, lab.astype(jnp.float32)], axis=0)
        gat = lax.dot_general(oh, rhs, nt,
                              preferred_element_type=jnp.float32)
        bout = gat[:, 0:5] + gat[:, 6:11]
        sout = gat[:, 5:6] + gat[:, 11:12]
        cout = gat[:, 12:13]

        kidx = lax.broadcasted_iota(jnp.int32, (_KPAD, 1),
                                    0).astype(jnp.float32)
        valid = kidx < jnp.minimum(nk, float(_MAXP))
        bb_ref[0] = jnp.where(valid, bout, -1.0)
        ss_ref[0] = jnp.where(valid, sout, -1.0)
        cc_ref[0] = jnp.where(valid, cout.astype(jnp.int32), -1)
        nk_ref[0] = nk.astype(jnp.int32).reshape(1, 1)


@jax.jit
def kernel(pred_boxes, pred_scores):
    boxesT = pred_boxes.transpose(0, 2, 1)
    scoresT = pred_scores.transpose(0, 2, 1)

    grid = (_B, _NB)
    out = pl.pallas_call(
        _nms_body,
        grid=grid,
        in_specs=[
            pl.BlockSpec((1, 5, _NPAD), lambda b, k: (b, 0, 0)),
            pl.BlockSpec((1, _C, _N), lambda b, k: (b, 0, 0)),
        ],
        out_specs=[
            pl.BlockSpec((1, 1, 1), lambda b, k: (b, 0, 0)),
            pl.BlockSpec((1, _KPAD, 5), lambda b, k: (b, 0, 0)),
            pl.BlockSpec((1, _KPAD, 1), lambda b, k: (b, 0, 0)),
            pl.BlockSpec((1, _KPAD, 1), lambda b, k: (b, 0, 0)),
        ],
        out_shape=[
            jax.ShapeDtypeStruct((_B, 1, 1), jnp.int32),
            jax.ShapeDtypeStruct((_B, _KPAD, 5), jnp.float32),
            jax.ShapeDtypeStruct((_B, _KPAD, 1), jnp.float32),
            jax.ShapeDtypeStruct((_B, _KPAD, 1), jnp.int32),
        ],
        scratch_shapes=[
            pltpu.VMEM((1, _NPAD), jnp.float32),
            pltpu.VMEM((1, _NPAD), jnp.float32),
            pltpu.VMEM((8, _N), jnp.float32),
        ],
        compiler_params=pltpu.CompilerParams(
            dimension_semantics=("parallel", "arbitrary")),
    )(boxesT, scoresT)
    nk3, b3, s3, c3 = out
    return (nk3.reshape(_B, 1), b3[:, :_MAXP, :],
            s3[:, :_MAXP, 0], c3[:, :_MAXP, 0])


# speedup vs baseline: 1.5614x; 1.0896x over previous
import functools

import jax
import jax.numpy as jnp
from jax import lax
from jax.experimental import pallas as pl
from jax.experimental.pallas import tpu as pltpu

_B = 4
_N = 2048
_C = 80
_MAXP = 300
_KPAD = 304
_CONF_TH = 0.25
_IOU_TH = 0.1
_SIGMA = 2.0
_EPS = 1e-7
_TR = 256
_NB = _N // _TR
_CW = 128
_NPAD = _N
_NEG = -1e30
_BIG = 1e30


def _row_params(bTc):
    xr = bTc[0:1, :]
    yr = bTc[1:2, :]
    wr = bTc[2:3, :]
    hr = bTc[3:4, :]
    rr = bTc[4:5, :]
    ar = wr * wr / 12.0
    br = hr * hr / 12.0
    cosr = jnp.cos(rr)
    sinr = jnp.sin(rr)
    Ar = ar * cosr * cosr + br * sinr * sinr
    Br = ar * sinr * sinr + br * cosr * cosr
    Cr = (ar - br) * cosr * sinr
    det = jnp.clip(Ar * Br - Cr * Cr, _EPS, None)
    rsdh = 0.5 / jnp.sqrt(det)
    return xr, yr, Ar, Br, Cr, rsdh


def _nms_body(bxT_ref, scT_ref, nk_ref, bb_ref, ss_ref, cc_ref,
              comp_ref, strict_ref, parm_ref):
    k = pl.program_id(1)

    @pl.when(k == 0)
    def _init():
        comp_ref[...] = jnp.zeros((1, _NPAD), jnp.float32)
        strict_ref[...] = jnp.full((1, _NPAD), _NEG, jnp.float32)
        xr, yr, Ar, Br, Cr, rsdh = _row_params(bxT_ref[0])
        parm_ref[...] = jnp.concatenate(
            [xr, yr, Ar, Br, Cr, rsdh, jnp.zeros((2, _N), jnp.float32)],
            axis=0)

    colstack = lax.transpose(parm_ref[:, pl.ds(k * _TR, _TR)], (1, 0))
    xc = colstack[:, 0:1]
    yc = colstack[:, 1:2]
    Ac = colstack[:, 2:3]
    Bc = colstack[:, 3:4]
    Cc = colstack[:, 4:5]
    rsdc = colstack[:, 5:6]

    def iou_tile(start, w):
        sl = pl.ds(start, w)
        xr = parm_ref[0:1, sl]
        yr = parm_ref[1:2, sl]
        Ar = parm_ref[2:3, sl]
        Br = parm_ref[3:4, sl]
        Cr = parm_ref[4:5, sl]
        rsdr = parm_ref[5:6, sl]
        As = Ac + Ar
        Bs = Bc + Br
        Cs = Cc + Cr
        denom = As * Bs - Cs * Cs + _EPS
        rden = 1.0 / denom
        dy = yc - yr
        dx = xc - xr
        t12 = (0.25 * (As * dy * dy + Bs * dx * dx)
               - 0.5 * Cs * dx * dy) * rden
        t3 = 0.5 * jnp.log(denom * (rsdc * rsdr) + _EPS)
        bd = jnp.maximum(t12 + t3, _EPS)
        hd = jnp.sqrt((1.0 + _EPS) - jnp.exp(-bd))
        return 1.0 - hd

    def iou_subtile(rlo, start, w):
        sl = pl.ds(start, w)
        xr = parm_ref[0:1, sl]
        yr = parm_ref[1:2, sl]
        Ar = parm_ref[2:3, sl]
        Br = parm_ref[3:4, sl]
        Cr = parm_ref[4:5, sl]
        rsdr = parm_ref[5:6, sl]
        As = Ac[rlo:rlo + _CW] + Ar
        Bs = Bc[rlo:rlo + _CW] + Br
        Cs = Cc[rlo:rlo + _CW] + Cr
        denom = As * Bs - Cs * Cs + _EPS
        rden = 1.0 / denom
        dy = yc[rlo:rlo + _CW] - yr
        dx = xc[rlo:rlo + _CW] - xr
        t12 = (0.25 * (As * dy * dy + Bs * dx * dx)
               - 0.5 * Cs * dx * dy) * rden
        t3 = 0.5 * jnp.log(denom * (rsdc[rlo:rlo + _CW] * rsdr) + _EPS)
        bd = jnp.maximum(t12 + t3, _EPS)
        hd = jnp.sqrt((1.0 + _EPS) - jnp.exp(-bd))
        return 1.0 - hd

    li = lax.broadcasted_iota(jnp.int32, (_CW, _CW), 0)
    lj = lax.broadcasted_iota(jnp.int32, (_CW, _CW), 1)
    dmask = li < lj

    def sub_diag(rlo):
        start = k * _TR + rlo
        dsl = pl.ds(start, _CW)
        iou = iou_subtile(rlo, start, _CW)
        comp_c = jnp.maximum(
            comp_ref[0:1, dsl],
            jnp.max(jnp.where(dmask, iou, 0.0), axis=0, keepdims=True))
        comp_ref[0:1, dsl] = comp_c
        compcol = jnp.max(jnp.where(li == lj, comp_c, _NEG), axis=1,
                          keepdims=True)
        csq_l = compcol * compcol
        term = jnp.where(dmask, iou * iou - csq_l, _NEG)
        strict_ref[0:1, dsl] = jnp.maximum(
            strict_ref[0:1, dsl], jnp.max(term, axis=0, keepdims=True))
        return csq_l

    csq0 = sub_diag(0)
    usl = pl.ds(k * _TR + _CW, _CW)
    iou_u = iou_subtile(0, k * _TR + _CW, _CW)
    comp_ref[0:1, usl] = jnp.maximum(
        comp_ref[0:1, usl], jnp.max(iou_u, axis=0, keepdims=True))
    strict_ref[0:1, usl] = jnp.maximum(
        strict_ref[0:1, usl],
        jnp.max(iou_u * iou_u - csq0, axis=0, keepdims=True))
    csq1 = sub_diag(_CW)
    csq = jnp.concatenate([csq0, csq1], axis=0)

    def half(start):
        sl = pl.ds(start, _CW)
        iou = iou_tile(start, _CW)
        comp_ref[0:1, sl] = jnp.maximum(
            comp_ref[0:1, sl], jnp.max(iou, axis=0, keepdims=True))
        strict_ref[0:1, sl] = jnp.maximum(
            strict_ref[0:1, sl],
            jnp.max(iou * iou - csq, axis=0, keepdims=True))

    def chunk(c, _):
        half(c * _TR)
        half(c * _TR + _CW)
        return 0

    lax.fori_loop(k + 1, _NB, chunk, 0)

    @pl.when(k == _NB - 1)
    def _epilogue():
        comp = comp_ref[0:1, :_N]
        x = comp
        sh = 1
        while sh < _N:
            shifted = jnp.concatenate(
                [x[:, sh:], jnp.full((1, sh), _BIG, jnp.float32)], axis=1)
            x = jnp.minimum(x, shifted)
            sh *= 2
        suffmin = x
        decayarg = jnp.maximum(strict_ref[0:1, :_N], -(suffmin * suffmin))
        decay = jnp.exp(-_SIGMA * decayarg)

        sc = scT_ref[0]
        confr = jnp.max(sc, axis=0, keepdims=True)
        idxc = lax.broadcasted_iota(jnp.int32, (_C, _N), 0)
        lab = jnp.min(jnp.where(sc == confr, idxc, 2 ** 30), axis=0,
                      keepdims=True)
        confr = jnp.where(confr < _CONF_TH, 0.0, confr)
        keep = (confr * decay) > _IOU_TH
        kf = keep.astype(jnp.float32)

        y = kf
        sh = 1
        while sh < _N:
            shifted = jnp.concatenate(
                [jnp.zeros((1, sh), jnp.float32), y[:, :_N - sh]], axis=1)
            y = y + shifted
            sh *= 2
        ck = y - kf
        nk = jnp.sum(kf)
        jr = lax.broadcasted_iota(jnp.int32, (1, _N), 1).astype(jnp.float32)
        pos = jnp.where(keep, ck, nk + (jr - ck))

        rowid = lax.broadcasted_iota(jnp.int32, (_KPAD, _N),
                                     0).astype(jnp.float32)
        oh = (rowid == pos).astype(jnp.float32)
        nt = (((1,), (1,)), ((), ()))
        bT = bxT_ref[0, :, : _N]
        vals = jnp.concatenate([bT, confr], axis=0)
        vhi = vals.astype(jnp.bfloat16).astype(jnp.float32)
        vlo = vals - vhi
        rhs = jnp.concatenate([vhi, vlo, lab.astype(jnp.float32)], axis=0)
        gat = lax.dot_general(oh, rhs, nt,
                              preferred_element_type=jnp.float32)
        bout = gat[:, 0:5] + gat[:, 6:11]
        sout = gat[:, 5:6] + gat[:, 11:12]
        cout = gat[:, 12:13]

        kidx = lax.broadcasted_iota(jnp.int32, (_KPAD, 1),
                                    0).astype(jnp.float32)
        valid = kidx < jnp.minimum(nk, float(_MAXP))
        bb_ref[0] = jnp.where(valid, bout, -1.0)
        ss_ref[0] = jnp.where(valid, sout, -1.0)
        cc_ref[0] = jnp.where(valid, cout.astype(jnp.int32), -1)
        nk_ref[0] = nk.astype(jnp.int32).reshape(1, 1)


@jax.jit
def kernel(pred_boxes, pred_scores):
    boxesT = pred_boxes.transpose(0, 2, 1)
    scoresT = pred_scores.transpose(0, 2, 1)

    grid = (_B, _NB)
    out = pl.pallas_call(
        _nms_body,
        grid=grid,
        in_specs=[
            pl.BlockSpec((1, 5, _NPAD), lambda b, k: (b, 0, 0)),
            pl.BlockSpec((1, _C, _N), lambda b, k: (b, 0, 0)),
        ],
        out_specs=[
            pl.BlockSpec((1, 1, 1), lambda b, k: (b, 0, 0)),
            pl.BlockSpec((1, _KPAD, 5), lambda b, k: (b, 0, 0)),
            pl.BlockSpec((1, _KPAD, 1), lambda b, k: (b, 0, 0)),
            pl.BlockSpec((1, _KPAD, 1), lambda b, k: (b, 0, 0)),
        ],
        out_shape=[
            jax.ShapeDtypeStruct((_B, 1, 1), jnp.int32),
            jax.ShapeDtypeStruct((_B, _KPAD, 5), jnp.float32),
            jax.ShapeDtypeStruct((_B, _KPAD, 1), jnp.float32),
            jax.ShapeDtypeStruct((_B, _KPAD, 1), jnp.int32),
        ],
        scratch_shapes=[
            pltpu.VMEM((1, _NPAD), jnp.float32),
            pltpu.VMEM((1, _NPAD), jnp.float32),
            pltpu.VMEM((8, _N), jnp.float32),
        ],
        compiler_params=pltpu.CompilerParams(
            dimension_semantics=("parallel", "arbitrary")),
    )(boxesT, scoresT)
    nk3, b3, s3, c3 = out
    return (nk3.reshape(_B, 1), b3[:, :_MAXP, :],
            s3[:, :_MAXP, 0], c3[:, :_MAXP, 0])
